# baseline (speedup 1.0000x reference)
import functools

import jax
import jax.numpy as jnp
from jax.experimental import pallas as pl
from jax.experimental.pallas import tpu as pltpu

N = 64
HID = 64
IN = 6
BT = 16
EPS = 1e-6


def _decoder_kernel(x_ref, wt_ref, WW_ref,
                    W2_ref, b2_ref, W3_ref, b3_ref,
                    Wres_ref, bres_ref,
                    Wo1_ref, bo1_ref, Wo2_ref, bo2_ref, Wo3_ref, bo3_ref,
                    out_ref, H_scr, agg_scr):
    NB = BT * N
    x = x_ref[:].reshape(NB, IN)
    pos = x[:, 0:3]
    vel = x[:, 3:6]

    n1 = jnp.sqrt(jnp.sum(vel * vel, axis=1, keepdims=True))
    e1 = vel / (n1 + EPS)
    e1x, e1y, e1z = e1[:, 0:1], e1[:, 1:2], e1[:, 2:3]
    rx, ry, rz = 0.12, 0.35, 0.93
    c2x = e1y * rz - e1z * ry
    c2y = e1z * rx - e1x * rz
    c2z = e1x * ry - e1y * rx
    e2 = jnp.concatenate([c2x, c2y, c2z], axis=1)
    n2 = jnp.sqrt(jnp.sum(e2 * e2, axis=1, keepdims=True))
    e2 = e2 / (n2 + EPS)
    e2x, e2y, e2z = e2[:, 0:1], e2[:, 1:2], e2[:, 2:3]
    c3x = e1y * e2z - e1z * e2y
    c3y = e1z * e2x - e1x * e2z
    c3z = e1x * e2y - e1y * e2x
    e3 = jnp.concatenate([c3x, c3y, c3z], axis=1)

    relp = jnp.concatenate([
        jnp.sum(e1 * pos, axis=1, keepdims=True),
        jnp.sum(e2 * pos, axis=1, keepdims=True),
        jnp.sum(e3 * pos, axis=1, keepdims=True)], axis=1)
    relv = jnp.concatenate([
        jnp.sum(e1 * vel, axis=1, keepdims=True),
        jnp.sum(e2 * vel, axis=1, keepdims=True),
        jnp.sum(e3 * vel, axis=1, keepdims=True)], axis=1)
    rel_feat = jnp.concatenate([relp, relv], axis=1)

    dot = functools.partial(jnp.dot, preferred_element_type=jnp.float32)

    z = lambda *s: jnp.zeros(s, jnp.float32)
    half = lambda a, lo: jnp.concatenate(
        [a[b * 64 + lo * 32:b * 64 + lo * 32 + 32] for b in range(BT)],
        axis=0)
    NH = NB // 2
    eye = (jax.lax.broadcasted_iota(jnp.int32, (6, 38), 1)
           == jax.lax.broadcasted_iota(jnp.int32, (6, 38), 0) + 6)
    eye2 = eye | (jax.lax.broadcasted_iota(jnp.int32, (6, 38), 1)
                  == jax.lax.broadcasted_iota(jnp.int32, (6, 38), 0) + 25)
    I6b = jnp.broadcast_to(eye2.astype(jnp.float32)[None], (NH, 6, 38))
    E3lo = jnp.stack([half(e1, 0), half(e2, 0), half(e3, 0)], axis=-1)
    E3hi = jnp.stack([half(e1, 1), half(e2, 1), half(e3, 1)], axis=-1)
    r1lo = jnp.concatenate([half(relp, 0), half(relv, 0)],
                           axis=1).reshape(NH, 1, 6)
    r1hi = jnp.concatenate([half(relp, 1), half(relv, 1)],
                           axis=1).reshape(NH, 1, 6)
    Mp = jnp.concatenate([E3lo, z(NH, 3, 16), E3hi, z(NH, 3, 16)], axis=2)
    Mv = jnp.concatenate([z(NH, 3, 3), E3lo, z(NH, 3, 16), E3hi,
                          z(NH, 3, 13)], axis=2)
    one = jnp.ones((NH, 1, 1), jnp.float32)
    crow = jnp.concatenate([z(NH, 1, 12), r1lo, one,
                            z(NH, 1, 12), r1hi, one], axis=2)
    M6 = jnp.concatenate([Mp, Mv, I6b, crow, z(NH, 3, 38)], axis=1)
    UW = dot(M6.reshape(NH * 16, 38), WW_ref[:]).reshape(NH, 16, 256)

    x16 = jnp.concatenate(
        [x, rel_feat, jnp.ones((NB, 1), jnp.float32), z(NB, 3)], axis=1)
    x16 = x16.astype(jnp.bfloat16)
    UWb = UW.astype(jnp.bfloat16)

    for b in range(BT):
        x16_b = x16[b * 64:(b + 1) * 64, :]
        for k in range(32):
            t = dot(x16_b, UWb[b * 32 + k])
            g_lo = 0.5 * jnp.tanh(0.5 * t[:, 64:128]) + 0.5
            g_hi = 0.5 * jnp.tanh(0.5 * t[:, 192:256]) + 0.5
            hg_lo = jnp.maximum(t[:, 0:64], 0.0) * g_lo
            hg_hi = jnp.maximum(t[:, 128:192], 0.0) * g_hi
            r = b * 64 + k
            H_scr[r * 64:(r + 1) * 64, :] = hg_lo.astype(jnp.bfloat16)
            rh = r + 32
            H_scr[rh * 64:(rh + 1) * 64, :] = hg_hi.astype(jnp.bfloat16)

    sws = []
    for b in range(BT):
        H2b = jnp.maximum(
            dot(H_scr[b * 4096:(b + 1) * 4096, :], W2_ref[:]) + b2_ref[:],
            0.0).astype(jnp.bfloat16)
        wtb = wt_ref[b]
        sws.append(jnp.sum(wtb, axis=1, keepdims=True))
        wtbb = wtb.astype(jnp.bfloat16)
        for j in range(64):
            agg_scr[b * 64 + j:b * 64 + j + 1, :] = dot(
                wtbb[j:j + 1, :], H2b[j * 64:(j + 1) * 64, :])
    sw = jnp.concatenate(sws, axis=0)
    agg = dot(agg_scr[:], W3_ref[:]) + sw * b3_ref[:]

    aug = agg + dot(rel_feat, Wres_ref[:]) + bres_ref[:]
    hh = jnp.maximum(dot(aug, Wo1_ref[:]) + bo1_ref[:], 0.0)
    hh = jnp.maximum(dot(hh, Wo2_ref[:]) + bo2_ref[:], 0.0)
    pred = dot(hh, Wo3_ref[:]) + bo3_ref[:]

    p0, p1, p2 = pred[:, 0:1], pred[:, 1:2], pred[:, 2:3]
    v0, v1, v2 = pred[:, 3:4], pred[:, 4:5], pred[:, 5:6]
    og_p = e1 * p0 + e2 * p1 + e3 * p2
    og_v = e1 * v0 + e2 * v1 + e3 * v2
    out_ref[:] = (x + jnp.concatenate([og_p, og_v], axis=1)).reshape(
        BT, N, IN)


def kernel(inputs, hidden, edges, W_res, b_res, W1, b1, Wp, bp,
           W2, b2, W3, b3, Wo1, bo1, Wo2, bo2, Wo3, bo3):
    B = inputs.shape[0]

    w = edges[..., 1]
    t = w.reshape(B, 63, 64)
    t = jnp.concatenate([t, jnp.zeros((B, 63, 1), jnp.float32)], axis=2)
    grid_ij = jnp.concatenate(
        [jnp.zeros((B, 1), jnp.float32), t.reshape(B, 63 * 65)],
        axis=1).reshape(B, 64, 64)
    wt = jnp.swapaxes(grid_ij, 1, 2) * (1.0 / 63.0)

    z364 = jnp.zeros((3, HID), jnp.float32)
    z664 = jnp.zeros((6, HID), jnp.float32)
    WW = jnp.concatenate([
        jnp.concatenate([W1[0:3] + W1[9:12], Wp[0:3] + Wp[3:6]], axis=1),
        jnp.concatenate([W1[3:6] + W1[6:9], z364], axis=1),
        jnp.concatenate([W1[12:18], z664], axis=1),
        -jnp.concatenate([W1[0:3], Wp[0:3]], axis=1),
        -jnp.concatenate([W1[3:6], z364], axis=1),
        jnp.concatenate([b1.reshape(1, -1), bp.reshape(1, -1)], axis=1),
    ], axis=0)
    z19 = jnp.zeros((19, 128), jnp.float32)
    WW2 = jnp.concatenate([
        jnp.concatenate([WW, z19], axis=1),
        jnp.concatenate([z19, WW], axis=1)], axis=0)
    r1 = lambda v: v.reshape(1, -1)

    full = lambda s: pl.BlockSpec(s, lambda b: (0,) * len(s))
    out = pl.pallas_call(
        _decoder_kernel,
        grid=(B // BT,),
        in_specs=[
            pl.BlockSpec((BT, N, IN), lambda b: (b, 0, 0)),
            pl.BlockSpec((BT, N, N), lambda b: (b, 0, 0)),
            full((38, 4 * HID)),
            full((HID, HID)), full((1, HID)),
            full((HID, HID)), full((1, HID)),
            full((IN, HID)), full((1, HID)),
            full((HID, HID)), full((1, HID)),
            full((HID, HID)), full((1, HID)),
            full((HID, IN)), full((1, IN)),
        ],
        out_specs=pl.BlockSpec((BT, N, IN), lambda b: (b, 0, 0)),
        out_shape=jax.ShapeDtypeStruct((B, N, IN), jnp.float32),
        scratch_shapes=[pltpu.VMEM((BT * N * N, HID), jnp.bfloat16),
                        pltpu.VMEM((BT * N, HID), jnp.float32)],
        compiler_params=pltpu.CompilerParams(
            dimension_semantics=("parallel",)),
    )(inputs, wt, WW2,
      W2.astype(jnp.bfloat16), r1(b2), W3, r1(b3), W_res, r1(b_res),
      Wo1, r1(bo1), Wo2, r1(bo2), Wo3, r1(bo3))
    return out


# speedup vs baseline: 1.0588x; 1.0588x over previous
import functools

import jax
import jax.numpy as jnp
from jax.experimental import pallas as pl
from jax.experimental.pallas import tpu as pltpu

N = 64
HID = 64
IN = 6
BT = 16
EPS = 1e-6


def _decoder_kernel(x_ref, wt_ref, WW_ref,
                    W2_ref, b2_ref, W3_ref, b3_ref,
                    Wres_ref, bres_ref,
                    Wo1_ref, bo1_ref, Wo2_ref, bo2_ref, Wo3_ref, bo3_ref,
                    out_ref, H_scr, agg_scr):
    NB = BT * N
    x = x_ref[:].reshape(NB, IN)
    pos = x[:, 0:3]
    vel = x[:, 3:6]

    n1 = jnp.sqrt(jnp.sum(vel * vel, axis=1, keepdims=True))
    e1 = vel / (n1 + EPS)
    e1x, e1y, e1z = e1[:, 0:1], e1[:, 1:2], e1[:, 2:3]
    rx, ry, rz = 0.12, 0.35, 0.93
    c2x = e1y * rz - e1z * ry
    c2y = e1z * rx - e1x * rz
    c2z = e1x * ry - e1y * rx
    e2 = jnp.concatenate([c2x, c2y, c2z], axis=1)
    n2 = jnp.sqrt(jnp.sum(e2 * e2, axis=1, keepdims=True))
    e2 = e2 / (n2 + EPS)
    e2x, e2y, e2z = e2[:, 0:1], e2[:, 1:2], e2[:, 2:3]
    c3x = e1y * e2z - e1z * e2y
    c3y = e1z * e2x - e1x * e2z
    c3z = e1x * e2y - e1y * e2x
    e3 = jnp.concatenate([c3x, c3y, c3z], axis=1)

    relp = jnp.concatenate([
        jnp.sum(e1 * pos, axis=1, keepdims=True),
        jnp.sum(e2 * pos, axis=1, keepdims=True),
        jnp.sum(e3 * pos, axis=1, keepdims=True)], axis=1)
    relv = jnp.concatenate([
        jnp.sum(e1 * vel, axis=1, keepdims=True),
        jnp.sum(e2 * vel, axis=1, keepdims=True),
        jnp.sum(e3 * vel, axis=1, keepdims=True)], axis=1)
    rel_feat = jnp.concatenate([relp, relv], axis=1)

    dot = functools.partial(jnp.dot, preferred_element_type=jnp.float32)

    z = lambda *s: jnp.zeros(s, jnp.float32)
    half = lambda a, lo: jnp.concatenate(
        [a[b * 64 + lo * 32:b * 64 + lo * 32 + 32] for b in range(BT)],
        axis=0)
    NH = NB // 2
    eye = (jax.lax.broadcasted_iota(jnp.int32, (6, 38), 1)
           == jax.lax.broadcasted_iota(jnp.int32, (6, 38), 0) + 6)
    eye2 = eye | (jax.lax.broadcasted_iota(jnp.int32, (6, 38), 1)
                  == jax.lax.broadcasted_iota(jnp.int32, (6, 38), 0) + 25)
    I6b = jnp.broadcast_to(eye2.astype(jnp.float32)[None], (NH, 6, 38))
    E3lo = jnp.stack([half(e1, 0), half(e2, 0), half(e3, 0)], axis=-1)
    E3hi = jnp.stack([half(e1, 1), half(e2, 1), half(e3, 1)], axis=-1)
    r1lo = jnp.concatenate([half(relp, 0), half(relv, 0)],
                           axis=1).reshape(NH, 1, 6)
    r1hi = jnp.concatenate([half(relp, 1), half(relv, 1)],
                           axis=1).reshape(NH, 1, 6)
    Mp = jnp.concatenate([E3lo, z(NH, 3, 16), E3hi, z(NH, 3, 16)], axis=2)
    Mv = jnp.concatenate([z(NH, 3, 3), E3lo, z(NH, 3, 16), E3hi,
                          z(NH, 3, 13)], axis=2)
    one = jnp.ones((NH, 1, 1), jnp.float32)
    crow = jnp.concatenate([z(NH, 1, 12), r1lo, one,
                            z(NH, 1, 12), r1hi, one], axis=2)
    M6 = jnp.concatenate([Mp, Mv, I6b, crow, z(NH, 3, 38)], axis=1)
    UW = dot(M6.reshape(NH * 16, 38), WW_ref[:]).reshape(NH, 16, 256)

    x16 = jnp.concatenate(
        [x, rel_feat, jnp.ones((NB, 1), jnp.float32), z(NB, 3)], axis=1)
    x16 = x16.astype(jnp.bfloat16)
    UWb = UW.astype(jnp.bfloat16)

    for b in range(BT):
        x16_b = x16[b * 64:(b + 1) * 64, :]
        for k in range(32):
            p = b * 32 + k
            t = dot(x16_b, UWb[p])
            g = 0.5 * jnp.tanh(0.5 * t[:, 128:256]) + 0.5
            hg = jnp.maximum(t[:, 0:128], 0.0) * g
            H_scr[p * 64:(p + 1) * 64, :] = hg.astype(jnp.bfloat16)

    lane1 = jax.lax.broadcasted_iota(jnp.int32, (1, 2 * HID), 1)
    lane32 = jax.lax.broadcasted_iota(jnp.int32, (32, 2 * HID), 1)
    sws = []
    for b in range(BT):
        H2p = jnp.maximum(
            dot(H_scr[b * 2048:(b + 1) * 2048, :], W2_ref[:]) + b2_ref[:],
            0.0).astype(jnp.bfloat16)
        wtpb = wt_ref[b]
        s2 = jnp.sum(wtpb, axis=2)
        sws.append(jnp.where(lane32 < HID, s2[:, 0:1], s2[:, 1:2]))
        wtpbb = wtpb.astype(jnp.bfloat16)
        for k in range(32):
            res = dot(wtpbb[k], H2p[k * 64:(k + 1) * 64, :])
            agg_scr[b * 32 + k:b * 32 + k + 1, :] = jnp.where(
                lane1 < HID, res[0:1, :], res[1:2, :])
    sw = jnp.concatenate(sws, axis=0)
    agg = dot(agg_scr[:], W3_ref[:]) + sw * b3_ref[:]

    rfp = jnp.concatenate(
        [jnp.concatenate([half(relp, 0), half(relv, 0)], axis=1),
         jnp.concatenate([half(relp, 1), half(relv, 1)], axis=1)],
        axis=1)
    aug = agg + dot(rfp, Wres_ref[:]) + bres_ref[:]
    hh = jnp.maximum(dot(aug, Wo1_ref[:]) + bo1_ref[:], 0.0)
    hh = jnp.maximum(dot(hh, Wo2_ref[:]) + bo2_ref[:], 0.0)
    predp = dot(hh, Wo3_ref[:]) + bo3_ref[:]

    pieces = []
    for b in range(BT):
        pieces.append(predp[b * 32:(b + 1) * 32, 0:IN])
        pieces.append(predp[b * 32:(b + 1) * 32, IN:2 * IN])
    pred = jnp.concatenate(pieces, axis=0)

    p0, p1, p2 = pred[:, 0:1], pred[:, 1:2], pred[:, 2:3]
    v0, v1, v2 = pred[:, 3:4], pred[:, 4:5], pred[:, 5:6]
    og_p = e1 * p0 + e2 * p1 + e3 * p2
    og_v = e1 * v0 + e2 * v1 + e3 * v2
    out_ref[:] = (x + jnp.concatenate([og_p, og_v], axis=1)).reshape(
        BT, N, IN)


def kernel(inputs, hidden, edges, W_res, b_res, W1, b1, Wp, bp,
           W2, b2, W3, b3, Wo1, bo1, Wo2, bo2, Wo3, bo3):
    B = inputs.shape[0]

    w = edges[..., 1]
    t = w.reshape(B, 63, 64)
    t = jnp.concatenate([t, jnp.zeros((B, 63, 1), jnp.float32)], axis=2)
    grid_ij = jnp.concatenate(
        [jnp.zeros((B, 1), jnp.float32), t.reshape(B, 63 * 65)],
        axis=1).reshape(B, 64, 64)
    wt = jnp.swapaxes(grid_ij, 1, 2) * (1.0 / 63.0)
    wtp = jnp.stack([wt[:, 0:32, :], wt[:, 32:64, :]], axis=2)

    z364 = jnp.zeros((3, HID), jnp.float32)
    z664 = jnp.zeros((6, HID), jnp.float32)
    WW = jnp.concatenate([
        jnp.concatenate([W1[0:3] + W1[9:12], Wp[0:3] + Wp[3:6]], axis=1),
        jnp.concatenate([W1[3:6] + W1[6:9], z364], axis=1),
        jnp.concatenate([W1[12:18], z664], axis=1),
        -jnp.concatenate([W1[0:3], Wp[0:3]], axis=1),
        -jnp.concatenate([W1[3:6], z364], axis=1),
        jnp.concatenate([b1.reshape(1, -1), bp.reshape(1, -1)], axis=1),
    ], axis=0)
    Wh, Wg = WW[:, 0:HID], WW[:, HID:]
    z19 = jnp.zeros((19, HID), jnp.float32)
    WW2 = jnp.concatenate([
        jnp.concatenate([Wh, z19, Wg, z19], axis=1),
        jnp.concatenate([z19, Wh, z19, Wg], axis=1)], axis=0)

    def bd(Wm):
        zz = jnp.zeros(Wm.shape, jnp.float32)
        return jnp.concatenate([
            jnp.concatenate([Wm, zz], axis=1),
            jnp.concatenate([zz, Wm], axis=1)], axis=0)

    t2 = lambda v: jnp.concatenate([v, v]).reshape(1, -1)

    full = lambda s: pl.BlockSpec(s, lambda b: (0,) * len(s))
    out = pl.pallas_call(
        _decoder_kernel,
        grid=(B // BT,),
        in_specs=[
            pl.BlockSpec((BT, N, IN), lambda b: (b, 0, 0)),
            pl.BlockSpec((BT, 32, 2, N), lambda b: (b, 0, 0, 0)),
            full((38, 4 * HID)),
            full((2 * HID, 2 * HID)), full((1, 2 * HID)),
            full((2 * HID, 2 * HID)), full((1, 2 * HID)),
            full((2 * IN, 2 * HID)), full((1, 2 * HID)),
            full((2 * HID, 2 * HID)), full((1, 2 * HID)),
            full((2 * HID, 2 * HID)), full((1, 2 * HID)),
            full((2 * HID, 2 * IN)), full((1, 2 * IN)),
        ],
        out_specs=pl.BlockSpec((BT, N, IN), lambda b: (b, 0, 0)),
        out_shape=jax.ShapeDtypeStruct((B, N, IN), jnp.float32),
        scratch_shapes=[pltpu.VMEM((BT * 32 * N, 2 * HID), jnp.bfloat16),
                        pltpu.VMEM((BT * 32, 2 * HID), jnp.float32)],
        compiler_params=pltpu.CompilerParams(
            dimension_semantics=("parallel",)),
    )(inputs, wtp, WW2,
      bd(W2).astype(jnp.bfloat16), t2(b2), bd(W3), t2(b3),
      bd(W_res), t2(b_res),
      bd(Wo1), t2(bo1), bd(Wo2), t2(bo2), bd(Wo3), t2(bo3))
    return out
